# baseline (device time: 114607 ns/iter reference)
import jax
import jax.numpy as jnp
from jax import lax
from jax.experimental import pallas as pl
from jax.experimental.pallas import tpu as pltpu

N_DEV = 4
B_LOC = 2
SQ = 512
SKV = 512
D_MODEL = 768
HG = 8
DH = 64
HD_LOC = HG * DH
BLK = 64


def kernel(x, Wq, K_ext, V_ext, Wo):
    my = lax.axis_index("i")

    K_t = lax.dynamic_slice_in_dim(K_ext, my * B_LOC, B_LOC, axis=0
                                   ).astype(jnp.bfloat16)
    V_t = lax.dynamic_slice_in_dim(V_ext, my * B_LOC, B_LOC, axis=0
                                   ).astype(jnp.bfloat16)

    x2 = x.reshape(B_LOC * SQ, D_MODEL)
    Wq = Wq.astype(jnp.bfloat16)
    Wo = Wo.astype(jnp.bfloat16)

    def body(x_ref, wq_ref, kt_ref, vt_ref, wo_ref, out_ref,
             wq_comm, wo_comm, wq_send, wq_recv, wo_send, wo_recv):
        me = lax.axis_index("i")
        left = (me + N_DEV - 1) % N_DEV
        right = (me + 1) % N_DEV

        ri = lax.broadcasted_iota(jnp.int32, (SQ, SKV), 0) // BLK
        ci = lax.broadcasted_iota(jnp.int32, (SQ, SKV), 1) // BLK
        bias = jnp.where(ci <= ri, 0.0, -1e9).astype(jnp.float32)

        x16 = x_ref[:, :].astype(jnp.bfloat16)

        def compute_group(g, wq_c, wo_c, first):
            Q = jnp.dot(x16, wq_c,
                        preferred_element_type=jnp.float32
                        ).astype(jnp.bfloat16)
            g0 = pl.multiple_of(g * HG, HG)
            parts = []
            for b in range(B_LOC):
                kg = kt_ref[b, :, pl.ds(g0, HG), :].reshape(SKV, HD_LOC)
                vg = vt_ref[b, :, pl.ds(g0, HG), :].reshape(SKV, HD_LOC)
                cs = []
                for h in range(HG):
                    q = Q[b * SQ:(b + 1) * SQ, h * DH:(h + 1) * DH]
                    k = kg[:, h * DH:(h + 1) * DH]
                    v = vg[:, h * DH:(h + 1) * DH]
                    s = lax.dot_general(
                        q, k, (((1,), (1,)), ((), ())),
                        preferred_element_type=jnp.float32,
                    )
                    p = jnp.exp(s * 0.125 + bias)
                    denom = jnp.sum(p, axis=1, keepdims=True)
                    c = jnp.dot(p.astype(jnp.bfloat16), v,
                                preferred_element_type=jnp.float32)
                    cs.append((c / denom).astype(jnp.bfloat16))
                parts.append(jnp.concatenate(cs, axis=1))
            C = jnp.concatenate(parts, axis=0)
            part = jnp.dot(C, wo_c, preferred_element_type=jnp.float32)
            if first:
                out_ref[:, :] = part
            else:
                out_ref[:, :] = out_ref[:, :] + part

        barrier_sem = pltpu.get_barrier_semaphore()
        for nbr in (left, right):
            pl.semaphore_signal(barrier_sem, inc=1, device_id=(nbr,),
                                device_id_type=pl.DeviceIdType.MESH)
        pl.semaphore_wait(barrier_sem, 2)

        def make_pair(src_q, src_o, slot):
            rq = pltpu.make_async_remote_copy(
                src_ref=src_q, dst_ref=wq_comm.at[slot],
                send_sem=wq_send.at[slot], recv_sem=wq_recv.at[slot],
                device_id=(right,), device_id_type=pl.DeviceIdType.MESH)
            ro = pltpu.make_async_remote_copy(
                src_ref=src_o, dst_ref=wo_comm.at[slot],
                send_sem=wo_send.at[slot], recv_sem=wo_recv.at[slot],
                device_id=(right,), device_id_type=pl.DeviceIdType.MESH)
            return rq, ro

        rq0, ro0 = make_pair(wq_ref, wo_ref, 0)
        rq0.start()
        ro0.start()
        rqs, ros = [rq0], [ro0]

        compute_group(me, wq_ref[:, :], wo_ref[:, :], first=True)

        for h in range(N_DEV - 1):
            rqs[h].wait_recv()
            ros[h].wait_recv()
            if h < N_DEV - 2:
                nq, no = make_pair(wq_comm.at[h], wo_comm.at[h], h + 1)
                nq.start()
                no.start()
                rqs.append(nq)
                ros.append(no)
            g = (me + (N_DEV - 1 - h)) % N_DEV
            compute_group(g, wq_comm[h], wo_comm[h], first=False)

        for h in range(N_DEV - 1):
            rqs[h].wait_send()
            ros[h].wait_send()

    out2 = pl.pallas_call(
        body,
        out_shape=jax.ShapeDtypeStruct((B_LOC * SQ, D_MODEL), jnp.float32),
        in_specs=[pl.BlockSpec(memory_space=pltpu.VMEM)] * 5,
        out_specs=pl.BlockSpec(memory_space=pltpu.VMEM),
        scratch_shapes=[
            pltpu.VMEM((N_DEV - 1, D_MODEL, HD_LOC), jnp.bfloat16),
            pltpu.VMEM((N_DEV - 1, HD_LOC, D_MODEL), jnp.bfloat16),
            pltpu.SemaphoreType.DMA((N_DEV - 1,)),
            pltpu.SemaphoreType.DMA((N_DEV - 1,)),
            pltpu.SemaphoreType.DMA((N_DEV - 1,)),
            pltpu.SemaphoreType.DMA((N_DEV - 1,)),
        ],
        compiler_params=pltpu.CompilerParams(
            collective_id=0,
            vmem_limit_bytes=100 * 1024 * 1024,
        ),
    )(x2, Wq, K_t, V_t, Wo)

    return out2.reshape(B_LOC, SQ, D_MODEL)


# device time: 98713 ns/iter; 1.1610x vs baseline; 1.1610x over previous
import jax
import jax.numpy as jnp
from jax import lax
from jax.experimental import pallas as pl
from jax.experimental.pallas import tpu as pltpu

N_DEV = 4
B_LOC = 2
SQ = 512
SKV = 512
D_MODEL = 768
HG = 8
DH = 64
HD_LOC = HG * DH
BLK = 64


def kernel(x, Wq, K_ext, V_ext, Wo):
    my = lax.axis_index("i")

    K_t = jnp.transpose(
        lax.dynamic_slice_in_dim(K_ext, my * B_LOC, B_LOC, axis=0
                                 ).astype(jnp.bfloat16), (2, 0, 1, 3))
    V_t = jnp.transpose(
        lax.dynamic_slice_in_dim(V_ext, my * B_LOC, B_LOC, axis=0
                                 ).astype(jnp.bfloat16), (2, 0, 1, 3))

    x2 = x.reshape(B_LOC * SQ, D_MODEL)
    Wq = Wq.astype(jnp.bfloat16)
    Wo = Wo.astype(jnp.bfloat16)

    def body(x_ref, wq_ref, kt_ref, vt_ref, wo_ref, out_ref,
             wq_comm, wo_comm, wq_send, wq_recv, wo_send, wo_recv):
        me = lax.axis_index("i")
        left = (me + N_DEV - 1) % N_DEV
        right = (me + 1) % N_DEV

        ri = lax.broadcasted_iota(jnp.int32, (SQ, SKV), 0) // BLK
        ci = lax.broadcasted_iota(jnp.int32, (SQ, SKV), 1) // BLK
        bias = jnp.where(ci <= ri, 0.0, -1e9).astype(jnp.float32)

        x16 = x_ref[:, :].astype(jnp.bfloat16)

        def compute_group(g, wq_c, wo_c, first):
            Q = jnp.dot(x16, wq_c,
                        preferred_element_type=jnp.float32
                        ).astype(jnp.bfloat16)
            parts = []
            for b in range(B_LOC):
                cs = []
                for h in range(HG):
                    gh = g * HG + h
                    q = Q[b * SQ:(b + 1) * SQ, h * DH:(h + 1) * DH]
                    k = kt_ref[pl.ds(gh, 1), b, :, :].reshape(SKV, DH)
                    v = vt_ref[pl.ds(gh, 1), b, :, :].reshape(SKV, DH)
                    s = lax.dot_general(
                        q, k, (((1,), (1,)), ((), ())),
                        preferred_element_type=jnp.float32,
                    )
                    p = jnp.exp(s * 0.125 + bias)
                    denom = jnp.sum(p, axis=1, keepdims=True)
                    c = jnp.dot(p.astype(jnp.bfloat16), v,
                                preferred_element_type=jnp.float32)
                    cs.append((c / denom).astype(jnp.bfloat16))
                parts.append(jnp.concatenate(cs, axis=1))
            C = jnp.concatenate(parts, axis=0)
            part = jnp.dot(C, wo_c, preferred_element_type=jnp.float32)
            if first:
                out_ref[:, :] = part
            else:
                out_ref[:, :] = out_ref[:, :] + part

        barrier_sem = pltpu.get_barrier_semaphore()
        for nbr in (left, right):
            pl.semaphore_signal(barrier_sem, inc=1, device_id=(nbr,),
                                device_id_type=pl.DeviceIdType.MESH)
        pl.semaphore_wait(barrier_sem, 2)

        def make_pair(src_q, src_o, slot):
            rq = pltpu.make_async_remote_copy(
                src_ref=src_q, dst_ref=wq_comm.at[slot],
                send_sem=wq_send.at[slot], recv_sem=wq_recv.at[slot],
                device_id=(right,), device_id_type=pl.DeviceIdType.MESH)
            ro = pltpu.make_async_remote_copy(
                src_ref=src_o, dst_ref=wo_comm.at[slot],
                send_sem=wo_send.at[slot], recv_sem=wo_recv.at[slot],
                device_id=(right,), device_id_type=pl.DeviceIdType.MESH)
            return rq, ro

        rq0, ro0 = make_pair(wq_ref, wo_ref, 0)
        rq0.start()
        ro0.start()
        rqs, ros = [rq0], [ro0]

        compute_group(me, wq_ref[:, :], wo_ref[:, :], first=True)

        for h in range(N_DEV - 1):
            rqs[h].wait_recv()
            ros[h].wait_recv()
            if h < N_DEV - 2:
                nq, no = make_pair(wq_comm.at[h], wo_comm.at[h], h + 1)
                nq.start()
                no.start()
                rqs.append(nq)
                ros.append(no)
            g = (me + (N_DEV - 1 - h)) % N_DEV
            compute_group(g, wq_comm[h], wo_comm[h], first=False)

        for h in range(N_DEV - 1):
            rqs[h].wait_send()
            ros[h].wait_send()

    out2 = pl.pallas_call(
        body,
        out_shape=jax.ShapeDtypeStruct((B_LOC * SQ, D_MODEL), jnp.float32),
        in_specs=[pl.BlockSpec(memory_space=pltpu.VMEM)] * 5,
        out_specs=pl.BlockSpec(memory_space=pltpu.VMEM),
        scratch_shapes=[
            pltpu.VMEM((N_DEV - 1, D_MODEL, HD_LOC), jnp.bfloat16),
            pltpu.VMEM((N_DEV - 1, HD_LOC, D_MODEL), jnp.bfloat16),
            pltpu.SemaphoreType.DMA((N_DEV - 1,)),
            pltpu.SemaphoreType.DMA((N_DEV - 1,)),
            pltpu.SemaphoreType.DMA((N_DEV - 1,)),
            pltpu.SemaphoreType.DMA((N_DEV - 1,)),
        ],
        compiler_params=pltpu.CompilerParams(
            collective_id=0,
            vmem_limit_bytes=100 * 1024 * 1024,
        ),
    )(x2, Wq, K_t, V_t, Wo)

    return out2.reshape(B_LOC, SQ, D_MODEL)


# device time: 75194 ns/iter; 1.5242x vs baseline; 1.3128x over previous
import jax
import jax.numpy as jnp
from jax import lax
from jax.experimental import pallas as pl
from jax.experimental.pallas import tpu as pltpu

N_DEV = 4
B_LOC = 2
SQ = 512
SKV = 512
D_MODEL = 768
HG = 8
DH = 64
HD_LOC = HG * DH
H2 = HD_LOC // 2
BLK = 64


def kernel(x, Wq, K_ext, V_ext, Wo):
    my = lax.axis_index("i")

    K_t = jnp.transpose(
        lax.dynamic_slice_in_dim(K_ext, my * B_LOC, B_LOC, axis=0
                                 ).astype(jnp.bfloat16), (2, 0, 1, 3))
    V_t = jnp.transpose(
        lax.dynamic_slice_in_dim(V_ext, my * B_LOC, B_LOC, axis=0
                                 ).astype(jnp.bfloat16), (2, 0, 1, 3))

    x2 = x.reshape(B_LOC * SQ, D_MODEL)
    Wq = Wq.astype(jnp.bfloat16)
    Wo = Wo.astype(jnp.bfloat16)

    def body(x_ref, wq_ref, kt_ref, vt_ref, wo_ref, out_ref,
             wq_comm, wo_comm, wq_send, wq_recv, wo_send, wo_recv):
        me = lax.axis_index("i")
        left = (me + N_DEV - 1) % N_DEV
        right = (me + 1) % N_DEV

        ri = lax.broadcasted_iota(jnp.int32, (SQ, SKV), 0) // BLK
        ci = lax.broadcasted_iota(jnp.int32, (SQ, SKV), 1) // BLK
        bias = jnp.where(ci <= ri, 0.0, -1e9).astype(jnp.float32)

        x16 = x_ref[:, :].astype(jnp.bfloat16)

        def compute_heads(gh0, nh, wq_c, wo_c, first):
            Q = jnp.dot(x16, wq_c,
                        preferred_element_type=jnp.float32
                        ).astype(jnp.bfloat16)
            parts = []
            for b in range(B_LOC):
                cs = []
                for h in range(nh):
                    gh = gh0 + h
                    q = Q[b * SQ:(b + 1) * SQ, h * DH:(h + 1) * DH]
                    k = kt_ref[pl.ds(gh, 1), b, :, :].reshape(SKV, DH)
                    v = vt_ref[pl.ds(gh, 1), b, :, :].reshape(SKV, DH)
                    s = lax.dot_general(
                        q, k, (((1,), (1,)), ((), ())),
                        preferred_element_type=jnp.float32,
                    )
                    p = jnp.exp(s * 0.125 + bias)
                    denom = jnp.sum(p, axis=1, keepdims=True)
                    c = jnp.dot(p.astype(jnp.bfloat16), v,
                                preferred_element_type=jnp.float32)
                    cs.append((c / denom).astype(jnp.bfloat16))
                parts.append(jnp.concatenate(cs, axis=1))
            C = jnp.concatenate(parts, axis=0)
            part = jnp.dot(C, wo_c, preferred_element_type=jnp.float32)
            if first:
                out_ref[:, :] = part
            else:
                out_ref[:, :] = out_ref[:, :] + part

        barrier_sem = pltpu.get_barrier_semaphore()
        for nbr in (left, right):
            pl.semaphore_signal(barrier_sem, inc=1, device_id=(nbr,),
                                device_id_type=pl.DeviceIdType.MESH)
        pl.semaphore_wait(barrier_sem, 2)

        def copy(src, dst, ssem, rsem, dev):
            return pltpu.make_async_remote_copy(
                src_ref=src, dst_ref=dst, send_sem=ssem, recv_sem=rsem,
                device_id=(dev,), device_id_type=pl.DeviceIdType.MESH)

        def start_pair(d, src_q, src_o, slot, dev):
            rq = copy(src_q, wq_comm.at[d, slot], wq_send.at[d, slot],
                      wq_recv.at[d, slot], dev)
            ro = copy(src_o, wo_comm.at[d, slot], wo_send.at[d, slot],
                      wo_recv.at[d, slot], dev)
            rq.start()
            ro.start()
            return [rq, ro]

        flights = [[None] * (N_DEV - 1) for _ in range(2)]
        flights[0][0] = start_pair(0, wq_ref.at[:, :H2], wo_ref.at[:H2, :],
                                   0, right)
        flights[1][0] = start_pair(1, wq_ref.at[:, H2:], wo_ref.at[H2:, :],
                                   0, left)

        compute_heads(me * HG, HG, wq_ref[:, :], wo_ref[:, :], first=True)

        for h in range(N_DEV - 1):
            for d in range(2):
                for r in flights[d][h]:
                    r.wait_recv()
                if h < N_DEV - 2:
                    dev = right if d == 0 else left
                    flights[d][h + 1] = start_pair(
                        d, wq_comm.at[d, h], wo_comm.at[d, h], h + 1, dev)
                if d == 0:
                    g = (me + (N_DEV - 1 - h)) % N_DEV
                    gh0 = g * HG
                else:
                    g = (me + h + 1) % N_DEV
                    gh0 = g * HG + H2 // DH
                compute_heads(gh0, H2 // DH, wq_comm[d, h], wo_comm[d, h],
                              first=False)

        for h in range(N_DEV - 1):
            for d in range(2):
                for r in flights[d][h]:
                    r.wait_send()

    out2 = pl.pallas_call(
        body,
        out_shape=jax.ShapeDtypeStruct((B_LOC * SQ, D_MODEL), jnp.float32),
        in_specs=[pl.BlockSpec(memory_space=pltpu.VMEM)] * 5,
        out_specs=pl.BlockSpec(memory_space=pltpu.VMEM),
        scratch_shapes=[
            pltpu.VMEM((2, N_DEV - 1, D_MODEL, H2), jnp.bfloat16),
            pltpu.VMEM((2, N_DEV - 1, H2, D_MODEL), jnp.bfloat16),
            pltpu.SemaphoreType.DMA((2, N_DEV - 1)),
            pltpu.SemaphoreType.DMA((2, N_DEV - 1)),
            pltpu.SemaphoreType.DMA((2, N_DEV - 1)),
            pltpu.SemaphoreType.DMA((2, N_DEV - 1)),
        ],
        compiler_params=pltpu.CompilerParams(
            collective_id=0,
            vmem_limit_bytes=100 * 1024 * 1024,
        ),
    )(x2, Wq, K_t, V_t, Wo)

    return out2.reshape(B_LOC, SQ, D_MODEL)
